# baseline (speedup 1.0000x reference)
import functools

import jax
import jax.numpy as jnp
from jax import lax
from jax.experimental import pallas as pl
from jax.experimental.pallas import tpu as pltpu
from jax.experimental.pallas import tpu_sc as plsc

_N = 100000
_D = 128
_S = 512
_NC = 2
_NS = 16
_NW = _NC * _NS
_K_SC = 48000
_RPW = _K_SC // _NW
_CHUNK = 125
_NCHUNK = _RPW // _CHUNK
_NBUF = 3
_SEGDMA = 1536
_SEGALLOC = 1552
_TCB = 4000
_TCG = (_N - _K_SC) // _TCB


def _sc_gated_pool(node_flat, segs, Wg2d, bg1):
  mesh = plsc.VectorSubcoreMesh(
      core_axis_name="c", subcore_axis_name="s",
      num_cores=_NC, num_subcores=_NS)

  @functools.partial(
      pl.kernel,
      out_type=jax.ShapeDtypeStruct((_NC, _S, _D), jnp.float32),
      mesh=mesh,
      scratch_types=[
          pltpu.VMEM((_SEGALLOC,), jnp.int32),
          pltpu.VMEM((_NBUF * _CHUNK * _D,), jnp.float32),
          pltpu.VMEM((_S, _D), jnp.float32),
          pltpu.VMEM((1, _D), jnp.float32),
          pltpu.VMEM((16,), jnp.float32),
          pltpu.VMEM((_S // 32, 32), jnp.int32),
          pltpu.VMEM_SHARED((_S, _D), jnp.float32),
          pltpu.SemaphoreType.DMA,
          pltpu.SemaphoreType.DMA,
          pltpu.SemaphoreType.DMA,
      ],
  )
  def k(node_hbm, seg_hbm, wg_hbm, bg_hbm, out_hbm,
        segv, buf, pool, gatev, bgb, idxv, shared, sem0, sem1, sem2):
    cid = lax.axis_index("c")
    sid = lax.axis_index("s")
    wid = sid * _NC + cid
    base = wid * _RPW
    a0 = jnp.minimum((base // 16) * 16, _N - _SEGDMA)
    soff = base - a0

    pltpu.sync_copy(wg_hbm, gatev)
    pltpu.sync_copy(bg_hbm, bgb.at[pl.ds(0, 1)])
    pltpu.sync_copy(seg_hbm.at[pl.ds(a0, _SEGDMA)], segv.at[pl.ds(0, _SEGDMA)])

    zeros16 = jnp.zeros((16,), jnp.float32)

    @plsc.parallel_loop(0, _S, 1, unroll=8)
    def zero_row(i):
      for kk in range(8):
        pool[i, pl.ds(16 * kk, 16)] = zeros16

    iota16 = lax.iota(jnp.int32, 16)
    for jj in range(_S // 32):
      idxv[jj, pl.ds(0, 16)] = iota16 + (32 * jj)
      idxv[jj, pl.ds(16, 16)] = iota16 + (32 * jj + 16)

    wg = [gatev[0, pl.ds(16 * kk, 16)] for kk in range(8)]
    iota0 = lax.iota(jnp.int32, 16) * 0
    bgv = bgb[pl.ds(0, 16)].at[iota0].get(
        mode="promise_in_bounds", unique_indices=False)
    perms = [lax.iota(jnp.int32, 16) ^ d for d in (8, 4, 2, 1)]

    def start(c, sem):
      boff = (c % _NBUF) * (_CHUNK * _D)
      pltpu.async_copy(
          node_hbm.at[pl.ds((base + c * _CHUNK) * _D, _CHUNK * _D)],
          buf.at[pl.ds(boff, _CHUNK * _D)], sem)

    def wait(sem):
      pltpu.make_async_copy(
          node_hbm.at[pl.ds(0, _CHUNK * _D)],
          buf.at[pl.ds(0, _CHUNK * _D)], sem).wait()

    def process(c, boff):
      segbase = soff + c * _CHUNK

      @plsc.parallel_loop(0, _CHUNK, 1, unroll=25)
      def row(r):
        rb = boff + r * _D
        xs = [buf[pl.ds(rb + 16 * kk, 16)] for kk in range(8)]
        acc = xs[0] * wg[0]
        for kk in range(1, 8):
          acc = acc + xs[kk] * wg[kk]
        for p in perms:
          acc = acc + acc.at[p].get(mode="promise_in_bounds", unique_indices=True)
        av = acc + bgv
        seg = segv[pl.ds(segbase + r, 16)][0]
        for kk in range(8):
          plsc.addupdate(pool.at[seg, pl.ds(16 * kk, 16)], av * xs[kk])

    @pl.when(sid == 0)
    def _():
      pltpu.sync_copy(pool, shared)

    plsc.subcore_barrier()

    start(0, sem0)
    start(1, sem1)

    def chunk_body(c, carry):
      m = c % _NBUF

      @pl.when(m == 0)
      def _():
        wait(sem0)

      @pl.when(m == 1)
      def _():
        wait(sem1)

      @pl.when(m == 2)
      def _():
        wait(sem2)

      @pl.when(c + 2 < _NCHUNK)
      def _():
        m2 = (c + 2) % _NBUF

        @pl.when(m2 == 0)
        def _():
          start(c + 2, sem0)

        @pl.when(m2 == 1)
        def _():
          start(c + 2, sem1)

        @pl.when(m2 == 2)
        def _():
          start(c + 2, sem2)

      process(c, m * (_CHUNK * _D))
      return carry

    lax.fori_loop(0, _NCHUNK, chunk_body, 0)

    s_first = segv[pl.ds(soff, 16)][0]
    s_last = segv[pl.ds(soff + _RPW - 1, 16)][0]

    def add_block(j, carry):
      pltpu.sync_copy(pool.at[pl.ds(j * 32, 32), :],
                      shared.at[idxv.at[j]], add=True)
      return carry

    lax.fori_loop(s_first // 32, s_last // 32 + 1, add_block, 0)
    plsc.subcore_barrier()

    @pl.when(sid == 0)
    def _():
      pltpu.sync_copy(shared, out_hbm.at[cid])

  return k(node_flat, segs, Wg2d, bg1)


def _tc_gated_pool(node_features, segs3d, Wg2d, bg2d):
  def body(x_ref, seg_ref, wg_ref, bg_ref, o_ref, acc_ref):
    g = pl.program_id(0)

    @pl.when(g == 0)
    def _():
      acc_ref[...] = jnp.zeros_like(acc_ref)

    x = x_ref[...]
    alpha = jnp.sum(x * wg_ref[...], axis=1, keepdims=True) + bg_ref[...]
    gated = (alpha * x).astype(jnp.bfloat16)
    seg = seg_ref[0, 0, :].astype(jnp.int16)
    oh = (seg[None, :] == lax.broadcasted_iota(
        jnp.int16, (_S, _TCB), 0)).astype(jnp.bfloat16)
    acc_ref[...] += lax.dot_general(
        oh, gated, (((1,), (0,)), ((), ())),
        preferred_element_type=jnp.float32)

    @pl.when(g == _TCG - 1)
    def _():
      o_ref[...] = acc_ref[...]

  blk0 = _K_SC // _TCB
  return pl.pallas_call(
      body,
      grid=(_TCG,),
      in_specs=[
          pl.BlockSpec((_TCB, _D), lambda g: (blk0 + g, 0)),
          pl.BlockSpec((1, 1, _TCB), lambda g: (blk0 + g, 0, 0)),
          pl.BlockSpec((1, _D), lambda g: (0, 0)),
          pl.BlockSpec((1, 1), lambda g: (0, 0)),
      ],
      out_specs=pl.BlockSpec((_S, _D), lambda g: (0, 0)),
      scratch_shapes=[pltpu.VMEM((_S, _D), jnp.float32)],
      out_shape=jax.ShapeDtypeStruct((_S, _D), jnp.float32),
  )(node_features, segs3d, Wg2d, bg2d)


def _tc_finish(sc_partials, tc_partial, Wp, bp2d):
  def body(p_ref, t_ref, wp_ref, bp_ref, o_ref):
    acc = p_ref[0] + p_ref[1] + t_ref[...]
    o_ref[...] = lax.dot_general(
        acc, wp_ref[...], (((1,), (1,)), ((), ())),
        preferred_element_type=jnp.float32) + bp_ref[...]

  return pl.pallas_call(
      body,
      out_shape=jax.ShapeDtypeStruct((_S, _D), jnp.float32),
  )(sc_partials, tc_partial, Wp, bp2d)


def kernel(node_features, batch_list, Wg, bg, Wp, bp):
  segs = batch_list.astype(jnp.int32)
  sc_partials = _sc_gated_pool(node_features.reshape(-1), segs,
                               Wg.astype(jnp.float32),
                               bg.astype(jnp.float32))
  tc_partial = _tc_gated_pool(
      node_features, segs.reshape(_N // _TCB, 1, _TCB),
      Wg.astype(jnp.float32), bg.reshape(1, 1).astype(jnp.float32))
  return _tc_finish(sc_partials, tc_partial, Wp, bp.reshape(1, _D))


# speedup vs baseline: 1.0202x; 1.0202x over previous
import functools

import jax
import jax.numpy as jnp
from jax import lax
from jax.experimental import pallas as pl
from jax.experimental.pallas import tpu as pltpu
from jax.experimental.pallas import tpu_sc as plsc

_N = 100000
_D = 128
_S = 512
_NC = 2
_NS = 16
_NW = _NC * _NS
_K_SC = 52000
_RPW = _K_SC // _NW
_CHUNK = 125
_NCHUNK = _RPW // _CHUNK
_NBUF = 3
_SEGDMA = 1664
_SEGALLOC = 1680
_TCB = 4000
_TCG = (_N - _K_SC) // _TCB


def _sc_gated_pool(node_flat, segs, Wg2d, bg1):
  mesh = plsc.VectorSubcoreMesh(
      core_axis_name="c", subcore_axis_name="s",
      num_cores=_NC, num_subcores=_NS)

  @functools.partial(
      pl.kernel,
      out_type=jax.ShapeDtypeStruct((_NC, _S, _D), jnp.float32),
      mesh=mesh,
      scratch_types=[
          pltpu.VMEM((_SEGALLOC,), jnp.int32),
          pltpu.VMEM((_NBUF * _CHUNK * _D,), jnp.float32),
          pltpu.VMEM((_S, _D), jnp.float32),
          pltpu.VMEM((1, _D), jnp.float32),
          pltpu.VMEM((16,), jnp.float32),
          pltpu.VMEM((_S // 32, 32), jnp.int32),
          pltpu.VMEM_SHARED((_S, _D), jnp.float32),
          pltpu.SemaphoreType.DMA,
          pltpu.SemaphoreType.DMA,
          pltpu.SemaphoreType.DMA,
      ],
  )
  def k(node_hbm, seg_hbm, wg_hbm, bg_hbm, out_hbm,
        segv, buf, pool, gatev, bgb, idxv, shared, sem0, sem1, sem2):
    cid = lax.axis_index("c")
    sid = lax.axis_index("s")
    wid = sid * _NC + cid
    base = wid * _RPW
    a0 = jnp.minimum((base // 16) * 16, _N - _SEGDMA)
    soff = base - a0

    pltpu.sync_copy(wg_hbm, gatev)
    pltpu.sync_copy(bg_hbm, bgb.at[pl.ds(0, 1)])
    pltpu.sync_copy(seg_hbm.at[pl.ds(a0, _SEGDMA)], segv.at[pl.ds(0, _SEGDMA)])

    zeros16 = jnp.zeros((16,), jnp.float32)

    @plsc.parallel_loop(0, _S, 1, unroll=8)
    def zero_row(i):
      for kk in range(8):
        pool[i, pl.ds(16 * kk, 16)] = zeros16

    iota16 = lax.iota(jnp.int32, 16)
    for jj in range(_S // 32):
      idxv[jj, pl.ds(0, 16)] = iota16 + (32 * jj)
      idxv[jj, pl.ds(16, 16)] = iota16 + (32 * jj + 16)

    wg = [gatev[0, pl.ds(16 * kk, 16)] for kk in range(8)]
    iota0 = lax.iota(jnp.int32, 16) * 0
    bgv = bgb[pl.ds(0, 16)].at[iota0].get(
        mode="promise_in_bounds", unique_indices=False)
    perms = [lax.iota(jnp.int32, 16) ^ d for d in (8, 4, 2, 1)]

    def start(c, sem):
      boff = (c % _NBUF) * (_CHUNK * _D)
      pltpu.async_copy(
          node_hbm.at[pl.ds((base + c * _CHUNK) * _D, _CHUNK * _D)],
          buf.at[pl.ds(boff, _CHUNK * _D)], sem)

    def wait(sem):
      pltpu.make_async_copy(
          node_hbm.at[pl.ds(0, _CHUNK * _D)],
          buf.at[pl.ds(0, _CHUNK * _D)], sem).wait()

    def process(c, boff):
      segbase = soff + c * _CHUNK

      @plsc.parallel_loop(0, _CHUNK, 1, unroll=25)
      def row(r):
        rb = boff + r * _D
        xs = [buf[pl.ds(rb + 16 * kk, 16)] for kk in range(8)]
        acc = xs[0] * wg[0]
        for kk in range(1, 8):
          acc = acc + xs[kk] * wg[kk]
        for p in perms:
          acc = acc + acc.at[p].get(mode="promise_in_bounds", unique_indices=True)
        av = acc + bgv
        seg = segv[pl.ds(segbase + r, 16)][0]
        for kk in range(8):
          plsc.addupdate(pool.at[seg, pl.ds(16 * kk, 16)], av * xs[kk])

    @pl.when(sid == 0)
    def _():
      pltpu.sync_copy(pool, shared)

    plsc.subcore_barrier()

    start(0, sem0)
    start(1, sem1)

    def chunk_body(c, carry):
      m = c % _NBUF

      @pl.when(m == 0)
      def _():
        wait(sem0)

      @pl.when(m == 1)
      def _():
        wait(sem1)

      @pl.when(m == 2)
      def _():
        wait(sem2)

      @pl.when(c + 2 < _NCHUNK)
      def _():
        m2 = (c + 2) % _NBUF

        @pl.when(m2 == 0)
        def _():
          start(c + 2, sem0)

        @pl.when(m2 == 1)
        def _():
          start(c + 2, sem1)

        @pl.when(m2 == 2)
        def _():
          start(c + 2, sem2)

      process(c, m * (_CHUNK * _D))
      return carry

    lax.fori_loop(0, _NCHUNK, chunk_body, 0)

    s_first = segv[pl.ds(soff, 16)][0]
    s_last = segv[pl.ds(soff + _RPW - 1, 16)][0]

    def add_block(j, carry):
      pltpu.sync_copy(pool.at[pl.ds(j * 32, 32), :],
                      shared.at[idxv.at[j]], add=True)
      return carry

    lax.fori_loop(s_first // 32, s_last // 32 + 1, add_block, 0)
    plsc.subcore_barrier()

    @pl.when(sid == 0)
    def _():
      pltpu.sync_copy(shared, out_hbm.at[cid])

  return k(node_flat, segs, Wg2d, bg1)


def _tc_gated_pool(node_features, segs3d, Wg2d, bg2d):
  def body(x_ref, seg_ref, wg_ref, bg_ref, o_ref, acc_ref):
    g = pl.program_id(0)

    @pl.when(g == 0)
    def _():
      acc_ref[...] = jnp.zeros_like(acc_ref)

    x = x_ref[...]
    alpha = jnp.sum(x * wg_ref[...], axis=1, keepdims=True) + bg_ref[...]
    gated = (alpha * x).astype(jnp.bfloat16)
    seg = seg_ref[0, 0, :].astype(jnp.int16)
    oh = (seg[None, :] == lax.broadcasted_iota(
        jnp.int16, (_S, _TCB), 0)).astype(jnp.bfloat16)
    acc_ref[...] += lax.dot_general(
        oh, gated, (((1,), (0,)), ((), ())),
        preferred_element_type=jnp.float32)

    @pl.when(g == _TCG - 1)
    def _():
      o_ref[...] = acc_ref[...]

  blk0 = _K_SC // _TCB
  return pl.pallas_call(
      body,
      grid=(_TCG,),
      in_specs=[
          pl.BlockSpec((_TCB, _D), lambda g: (blk0 + g, 0)),
          pl.BlockSpec((1, 1, _TCB), lambda g: (blk0 + g, 0, 0)),
          pl.BlockSpec((1, _D), lambda g: (0, 0)),
          pl.BlockSpec((1, 1), lambda g: (0, 0)),
      ],
      out_specs=pl.BlockSpec((_S, _D), lambda g: (0, 0)),
      scratch_shapes=[pltpu.VMEM((_S, _D), jnp.float32)],
      out_shape=jax.ShapeDtypeStruct((_S, _D), jnp.float32),
  )(node_features, segs3d, Wg2d, bg2d)


def _tc_finish(sc_partials, tc_partial, Wp, bp2d):
  def body(p_ref, t_ref, wp_ref, bp_ref, o_ref):
    acc = p_ref[0] + p_ref[1] + t_ref[...]
    o_ref[...] = lax.dot_general(
        acc, wp_ref[...], (((1,), (1,)), ((), ())),
        preferred_element_type=jnp.float32) + bp_ref[...]

  return pl.pallas_call(
      body,
      out_shape=jax.ShapeDtypeStruct((_S, _D), jnp.float32),
  )(sc_partials, tc_partial, Wp, bp2d)


def kernel(node_features, batch_list, Wg, bg, Wp, bp):
  segs = batch_list.astype(jnp.int32)
  sc_partials = _sc_gated_pool(node_features.reshape(-1), segs,
                               Wg.astype(jnp.float32),
                               bg.astype(jnp.float32))
  tc_partial = _tc_gated_pool(
      node_features, segs.reshape(_N // _TCB, 1, _TCB),
      Wg.astype(jnp.float32), bg.reshape(1, 1).astype(jnp.float32))
  return _tc_finish(sc_partials, tc_partial, Wp, bp.reshape(1, _D))
